# baseline (speedup 1.0000x reference)
import jax
import jax.numpy as jnp
from jax.experimental import pallas as pl
from jax.experimental.pallas import tpu as pltpu

H = 128
B = 8


def _mlp_body(*refs):
    x_ref = refs[0]
    o_ref = refs[-1]
    wrefs = refs[1:-1]
    x = x_ref[:]
    n_groups = (len(wrefs) - 2) // 4
    for i in range(n_groups):
        Wt, b, g, bb = wrefs[4 * i:4 * i + 4]
        h = jnp.dot(x, Wt[:], preferred_element_type=jnp.float32) + b[:]
        m = jnp.mean(h, axis=-1, keepdims=True)
        v = jnp.mean((h - m) ** 2, axis=-1, keepdims=True)
        h = (h - m) / jnp.sqrt(v + 1e-5) * g[:] + bb[:]
        x = jnp.maximum(h, 0.0)
    Wft, bf = wrefs[-2], wrefs[-1]
    o_ref[:] = jnp.dot(x, Wft[:], preferred_element_type=jnp.float32) + bf[:]


def _mlp_pallas(x, p):
    args = [x]
    for g in p['groups']:
        args += [g['W'].T, g['b'][None, :], g['g'][None, :], g['bb'][None, :]]
    Wf = p['final']['W']
    Wft = jnp.zeros((H, H), jnp.float32).at[:, :2].set(Wf.T)
    bf = jnp.zeros((1, H), jnp.float32).at[0, :2].set(p['final']['b'])
    args += [Wft, bf]
    out = pl.pallas_call(
        _mlp_body,
        out_shape=jax.ShapeDtypeStruct((B, H), jnp.float32),
    )(*args)
    return out[:, :2]


def _linear(x, W, b):
    return x @ W.T + b


def _layer_norm(x, g, b, eps=1e-5):
    m = jnp.mean(x, axis=-1, keepdims=True)
    v = jnp.mean((x - m) ** 2, axis=-1, keepdims=True)
    return (x - m) / jnp.sqrt(v + eps) * g + b


def _seg_max(data, seg, num):
    out = jax.ops.segment_max(data, seg, num_segments=num)
    return jnp.where(jnp.isfinite(out), out, 0.0)


def _residual_mpnn(x, src, dst, n, p):
    agg = _seg_max(x[src], dst, n)
    out = _layer_norm(_linear(agg, p['lin_W'], p['lin_b']), p['ln_g'], p['ln_b'])
    return jax.nn.relu(out + x)


def _weighted_mpnn(x, src, dst, ea, n, p):
    agg = jax.ops.segment_sum(x[src] * ea[:, None], dst, num_segments=n)
    out = _layer_norm(_linear(agg, p['lin_W'], p['lin_b']), p['ln_g'], p['ln_b'])
    return jax.nn.relu(out + x)


def _three_hop(x, src, dst, n, p):
    x = jax.nn.relu(_linear(x, p['proj_W'], p['proj_b']))
    hops = []
    for i in range(3):
        x = _residual_mpnn(x, src, dst, n, p['hops'][i])
        hops.append(x)
    fused = jnp.zeros_like(x)
    for i in range(3):
        fused = fused + hops[i] * jax.nn.sigmoid(p['gates'][i])
    return fused


def _single_unit(x, src, dst, ea, batch, bsz, p):
    n = x.shape[0]
    nf = _three_hop(x, src, dst, n, p['gnn'])
    poly = (ea == 3) & (batch[src] == batch[dst])
    msg = jnp.where(poly[:, None], nf[src], -jnp.inf)
    agg = jax.ops.segment_max(msg, dst, num_segments=n)
    agg = jnp.where(jnp.isfinite(agg), agg, 0.0)
    pf = p['final']
    subf = jax.nn.relu(_layer_norm(_linear(agg, pf['lin_W'], pf['lin_b']), pf['ln_g'], pf['ln_b']) + nf)
    sink_data = jnp.where(poly[:, None], subf[dst], -jnp.inf)
    vs2 = jax.ops.segment_max(sink_data, batch[dst], num_segments=bsz)
    vs2 = jnp.where(jnp.isfinite(vs2), vs2, 1e-4)
    keep = ~(x[:, 0] > 0.1)
    keep_data = jnp.where(keep[:, None], nf, -jnp.inf)
    rv = jax.ops.segment_max(keep_data, batch, num_segments=bsz)
    rv = jnp.where(jnp.isfinite(rv), rv, 1e-4)
    return jnp.maximum(vs2, rv)


def _large_block(x, src, dst, ea, batch, bsz, p):
    n = x.shape[0]
    x = jax.nn.relu(_linear(x, p['proj_W'], p['proj_b']))
    for i in range(3):
        x = _weighted_mpnn(x, src, dst, ea, n, p['hops'][i])
    pooled = _seg_max(x, batch, bsz)
    return jax.nn.relu(_linear(pooled, p['pool_W'], p['pool_b']))


def kernel(x_small, edge_index_small, edge_attr_small, batch_small, x_large,
           edge_index_large, edge_attr_large, batch_large, params):
    src_s, dst_s = edge_index_small[0], edge_index_small[1]
    feats = []
    for name in ('outer', 'middle', 'inner'):
        feats.append(_single_unit(x_small, src_s, dst_s, edge_attr_small,
                                  batch_small, B, params[name]))
    feats.append(_large_block(x_large, edge_index_large[0], edge_index_large[1],
                              edge_attr_large, batch_large, B, params['large']))
    return _mlp_pallas(jnp.concatenate(feats, axis=1), params['mlp'])


# speedup vs baseline: 1.1489x; 1.1489x over previous
import functools

import jax
import jax.numpy as jnp
from jax import lax
from jax.experimental import pallas as pl
from jax.experimental.pallas import tpu as pltpu
from jax.experimental.pallas import tpu_sc as plsc

H = 128
B = 8
NW = 32
NPT = 320
N_PAD = NW * NPT
CH = 2048
GB = 256
NCC = 8


def _sload(ref, idx):
    return ref[pl.ds(idx, 16)][0]


def _seg_body(is_sum, nchunks, *refs):
    if is_sum:
        (x_hbm, srcs_hbm, dsts_hbm, w_hbm, ebnd_hbm, out_hbm,
         ebnd_v, srcs_v, dsts_v, w_v, rows_v, acc_v, sem) = refs
    else:
        (x_hbm, srcs_hbm, dsts_hbm, ebnd_hbm, out_hbm,
         ebnd_v, srcs_v, dsts_v, rows_v, acc_v, sem) = refs
        w_hbm = w_v = None
    wid = lax.axis_index("s") * 2 + lax.axis_index("c")
    nbase = wid * NPT
    pltpu.sync_copy(ebnd_hbm, ebnd_v)
    e0 = _sload(ebnd_v, wid)
    e1 = _sload(ebnd_v, wid + 1)

    zero = jnp.zeros((16,), jnp.float32)

    def zrow(r, carry):
        for c in range(NCC):
            acc_v[r, pl.ds(c * 16, 16)] = zero
        return carry

    lax.fori_loop(0, NPT, zrow, 0)

    ci_lo = e0 // CH
    ci_hi = (e1 + CH - 1) // CH

    def chunk_body(ci, carry):
        cb = ci * CH
        pltpu.sync_copy(srcs_hbm.at[pl.ds(cb, CH)], srcs_v.at[pl.ds(0, CH)])
        pltpu.sync_copy(dsts_hbm.at[pl.ds(cb, CH)], dsts_v.at[pl.ds(0, CH)])
        if is_sum:
            pltpu.sync_copy(w_hbm.at[pl.ds(cb, CH)], w_v.at[pl.ds(0, CH)])
        j_lo = jnp.maximum((e0 - cb) // GB, 0)
        j_hi = jnp.minimum((e1 - cb + GB - 1) // GB, CH // GB)

        def batch_body(j, carry2):
            bb = j * GB
            pltpu.async_copy(x_hbm.at[srcs_v.at[pl.ds(bb, GB)]], rows_v,
                             sem).wait()
            lo = jnp.maximum(e0, cb + bb)
            hi = jnp.minimum(e1, cb + bb + GB)

            def edge_body(e, carry3):
                k = e - cb
                r = k - bb
                a = _sload(dsts_v, k) - nbase
                if is_sum:
                    wgt = _sload(w_v, k)
                    for c in range(NCC):
                        cs = pl.ds(c * 16, 16)
                        acc_v[a, cs] = acc_v[a, cs] + rows_v[r, cs] * wgt
                else:
                    for c in range(NCC):
                        cs = pl.ds(c * 16, 16)
                        acc_v[a, cs] = jnp.maximum(acc_v[a, cs],
                                                   rows_v[r, cs])
                return carry3

            lax.fori_loop(lo, hi, edge_body, 0)
            return carry2

        lax.fori_loop(j_lo, j_hi, batch_body, 0)
        return carry

    lax.fori_loop(ci_lo, ci_hi, chunk_body, 0)
    pltpu.sync_copy(acc_v, out_hbm.at[pl.ds(nbase, NPT)])


def _make_seg_kernel(is_sum, e_pad):
    nchunks = e_pad // CH
    mesh = plsc.VectorSubcoreMesh(core_axis_name="c", subcore_axis_name="s")
    scratch = [
        pltpu.VMEM((48,), jnp.int32),
        pltpu.VMEM((CH + 16,), jnp.int32),
        pltpu.VMEM((CH + 16,), jnp.int32),
    ]
    if is_sum:
        scratch.append(pltpu.VMEM((CH + 16,), jnp.float32))
    scratch += [
        pltpu.VMEM((GB, H), jnp.float32),
        pltpu.VMEM((NPT, H), jnp.float32),
        pltpu.SemaphoreType.DMA,
    ]
    return pl.kernel(
        functools.partial(_seg_body, is_sum, nchunks),
        out_type=jax.ShapeDtypeStruct((N_PAD, H), jnp.float32),
        mesh=mesh,
        scratch_types=scratch,
    )


def _sc_prep(edge_index, w=None):
    src, dst = edge_index[0], edge_index[1]
    e = src.shape[0]
    e_pad = ((e + CH - 1) // CH) * CH
    perm = jnp.argsort(dst)
    srcs = jnp.pad(src[perm], (0, e_pad - e))
    dsts = jnp.pad(dst[perm], (0, e_pad - e), constant_values=N_PAD)
    bounds = jnp.arange(NW + 1, dtype=jnp.int32) * NPT
    ebnd = jnp.searchsorted(dsts, bounds).astype(jnp.int32)
    ebnd = jnp.pad(ebnd, (0, 48 - NW - 1))
    out = {'srcs': srcs, 'dsts': dsts, 'ebnd': ebnd, 'e_pad': e_pad}
    if w is not None:
        out['w'] = jnp.pad(w[perm], (0, e_pad - e))
    return out


def _sc_segmax(x, prep):
    k = _make_seg_kernel(False, prep['e_pad'])
    return k(x, prep['srcs'], prep['dsts'], prep['ebnd'])[:x.shape[0]]


def _sc_segsum(x, prep):
    k = _make_seg_kernel(True, prep['e_pad'])
    return k(x, prep['srcs'], prep['dsts'], prep['w'], prep['ebnd'])[:x.shape[0]]


def _mlp_body(*refs):
    x_ref = refs[0]
    o_ref = refs[-1]
    wrefs = refs[1:-1]
    x = x_ref[:]
    n_groups = (len(wrefs) - 2) // 4
    for i in range(n_groups):
        Wt, b, g, bb = wrefs[4 * i:4 * i + 4]
        h = jnp.dot(x, Wt[:], preferred_element_type=jnp.float32) + b[:]
        m = jnp.mean(h, axis=-1, keepdims=True)
        v = jnp.mean((h - m) ** 2, axis=-1, keepdims=True)
        h = (h - m) / jnp.sqrt(v + 1e-5) * g[:] + bb[:]
        x = jnp.maximum(h, 0.0)
    Wft, bf = wrefs[-2], wrefs[-1]
    o_ref[:] = jnp.dot(x, Wft[:], preferred_element_type=jnp.float32) + bf[:]


def _mlp_pallas(x, p):
    args = [x]
    for g in p['groups']:
        args += [g['W'].T, g['b'][None, :], g['g'][None, :], g['bb'][None, :]]
    Wf = p['final']['W']
    Wft = jnp.zeros((H, H), jnp.float32).at[:, :2].set(Wf.T)
    bf = jnp.zeros((1, H), jnp.float32).at[0, :2].set(p['final']['b'])
    args += [Wft, bf]
    out = pl.pallas_call(
        _mlp_body,
        out_shape=jax.ShapeDtypeStruct((B, H), jnp.float32),
    )(*args)
    return out[:, :2]


def _linear(x, W, b):
    return x @ W.T + b


def _layer_norm(x, g, b, eps=1e-5):
    m = jnp.mean(x, axis=-1, keepdims=True)
    v = jnp.mean((x - m) ** 2, axis=-1, keepdims=True)
    return (x - m) / jnp.sqrt(v + eps) * g + b


def _residual_mpnn(x, prep, p):
    agg = _sc_segmax(x, prep)
    out = _layer_norm(_linear(agg, p['lin_W'], p['lin_b']), p['ln_g'], p['ln_b'])
    return jax.nn.relu(out + x)


def _weighted_mpnn(x, prep, p):
    agg = _sc_segsum(x, prep)
    out = _layer_norm(_linear(agg, p['lin_W'], p['lin_b']), p['ln_g'], p['ln_b'])
    return jax.nn.relu(out + x)


def _three_hop(x, prep, p):
    x = jax.nn.relu(_linear(x, p['proj_W'], p['proj_b']))
    hops = []
    for i in range(3):
        x = _residual_mpnn(x, prep, p['hops'][i])
        hops.append(x)
    fused = jnp.zeros_like(x)
    for i in range(3):
        fused = fused + hops[i] * jax.nn.sigmoid(p['gates'][i])
    return fused


def _single_unit(x, prep, src, dst, ea, batch, bsz, p):
    n = x.shape[0]
    nf = _three_hop(x, prep, p['gnn'])
    poly = (ea == 3) & (batch[src] == batch[dst])
    msg = jnp.where(poly[:, None], nf[src], -jnp.inf)
    agg = jax.ops.segment_max(msg, dst, num_segments=n)
    agg = jnp.where(jnp.isfinite(agg), agg, 0.0)
    pf = p['final']
    subf = jax.nn.relu(_layer_norm(_linear(agg, pf['lin_W'], pf['lin_b']),
                                   pf['ln_g'], pf['ln_b']) + nf)
    sink_data = jnp.where(poly[:, None], subf[dst], -jnp.inf)
    vs2 = jax.ops.segment_max(sink_data, batch[dst], num_segments=bsz)
    vs2 = jnp.where(jnp.isfinite(vs2), vs2, 1e-4)
    keep = ~(x[:, 0] > 0.1)
    keep_data = jnp.where(keep[:, None], nf, -jnp.inf)
    rv = jax.ops.segment_max(keep_data, batch, num_segments=bsz)
    rv = jnp.where(jnp.isfinite(rv), rv, 1e-4)
    return jnp.maximum(vs2, rv)


def _large_block(x, prep, batch, bsz, p):
    x = jax.nn.relu(_linear(x, p['proj_W'], p['proj_b']))
    for i in range(3):
        x = _weighted_mpnn(x, prep, p['hops'][i])
    pooled = jax.ops.segment_max(x, batch, num_segments=bsz)
    pooled = jnp.where(jnp.isfinite(pooled), pooled, 0.0)
    return jax.nn.relu(_linear(pooled, p['pool_W'], p['pool_b']))


def kernel(x_small, edge_index_small, edge_attr_small, batch_small, x_large,
           edge_index_large, edge_attr_large, batch_large, params):
    src_s, dst_s = edge_index_small[0], edge_index_small[1]
    prep_s = _sc_prep(edge_index_small)
    prep_l = _sc_prep(edge_index_large, w=edge_attr_large)
    feats = []
    for name in ('outer', 'middle', 'inner'):
        feats.append(_single_unit(x_small, prep_s, src_s, dst_s,
                                  edge_attr_small, batch_small, B,
                                  params[name]))
    feats.append(_large_block(x_large, prep_l, batch_large, B,
                              params['large']))
    return _mlp_pallas(jnp.concatenate(feats, axis=1), params['mlp'])
